# Initial kernel scaffold; baseline (speedup 1.0000x reference)
import jax
import jax.numpy as jnp
from jax.experimental import pallas as pl


def kernel(x, edge_index, batch, W1, a_src1, a_dst1, b1, g1, be1, W2, a_src2, a_dst2, b2, g2, be2, W3, a_src3, a_dst3, b3, g3, be3, Wp, bp):
    raise NotImplementedError("write your pallas kernel here")


# speedup vs baseline: 20.6404x; 20.6404x over previous
import functools

import jax
import jax.numpy as jnp
from jax import lax
from jax.experimental import pallas as pl
from jax.experimental.pallas import tpu as pltpu
from jax.experimental.pallas import tpu_sc as plsc

N = 10000
E = 320000
D_IN = 128
HID = 64
HEADS = 4
D_OUT = 128

NSC = 2
NSUB = 16
NPAD = 10240
TBL = 10016
CH = 128
C = 48
CPS = 417
EPAD = CPS * NSUB * C
ROWS_PER_SUB = NPAD // NSUB
RC = 32
RCH = ROWS_PER_SUB // RC
BLK = 1024


def _k1_body(x_ref, w_ref, a_ref, h_ref, al_ref):
  h = jnp.dot(x_ref[...], w_ref[...], preferred_element_type=jnp.float32)
  h_ref[0] = h[:, :CH]
  h_ref[1] = h[:, CH:]
  al_ref[...] = lax.dot_general(
      a_ref[...], h, (((0,), (1,)), ((), ())),
      preferred_element_type=jnp.float32)


def _kmid_body(d_out, gin_ref, b_ref, g_ref, be_ref, w_ref, a_ref, h_ref,
               al_ref):
  t = jnp.concatenate([gin_ref[0], gin_ref[1]], axis=-1) + b_ref[...]
  m = jnp.mean(t, axis=-1, keepdims=True)
  v = jnp.mean((t - m) ** 2, axis=-1, keepdims=True)
  t = (t - m) * lax.rsqrt(v + 1e-5) * g_ref[...] + be_ref[...]
  t = jnp.maximum(t, 0.0)
  h = jnp.dot(t, w_ref[...], preferred_element_type=jnp.float32)
  if d_out == 256:
    h_ref[0] = h[:, :CH]
    h_ref[1] = h[:, CH:]
  else:
    z = jnp.zeros_like(h[:, :64])
    h_ref[0] = jnp.concatenate([h[:, :64], z], axis=-1)
    h_ref[1] = jnp.concatenate([h[:, 64:], z], axis=-1)
  al_ref[...] = lax.dot_general(
      a_ref[...], h, (((0,), (1,)), ((), ())),
      preferred_element_type=jnp.float32)


def _k4_body(gin_ref, b_ref, g_ref, be_ref, wp_ref, bp_ref, o_ref):
  t = jnp.concatenate([gin_ref[0][:, :64], gin_ref[1][:, :64]], axis=-1)
  t = t + b_ref[...]
  m = jnp.mean(t, axis=-1, keepdims=True)
  v = jnp.mean((t - m) ** 2, axis=-1, keepdims=True)
  t = (t - m) * lax.rsqrt(v + 1e-5) * g_ref[...] + be_ref[...]
  t = jnp.maximum(t, 0.0)
  mask = lax.broadcasted_iota(jnp.int32, (NPAD, 1), 0) < N
  tm = jnp.where(mask, t, 0.0)
  mean = jnp.sum(tm, axis=0, keepdims=True) * (1.0 / N)
  tmax = jnp.max(jnp.where(mask, t, -1e30), axis=0, keepdims=True)
  feat = jnp.concatenate([mean, tmax], axis=-1)
  o_ref[...] = jnp.dot(feat, wp_ref[...],
                       preferred_element_type=jnp.float32) + bp_ref[...]


def _mk_k1(d_in, d_out, n_al):
  return pl.pallas_call(
      _k1_body,
      grid=(NPAD // BLK,),
      in_specs=[
          pl.BlockSpec((BLK, d_in), lambda i: (i, 0)),
          pl.BlockSpec((d_in, d_out), lambda i: (0, 0)),
          pl.BlockSpec((d_out, n_al), lambda i: (0, 0)),
      ],
      out_specs=[
          pl.BlockSpec((2, BLK, CH), lambda i: (0, i, 0)),
          pl.BlockSpec((n_al, BLK), lambda i: (0, i)),
      ],
      out_shape=[
          jax.ShapeDtypeStruct((2, NPAD, CH), jnp.float32),
          jax.ShapeDtypeStruct((n_al, NPAD), jnp.float32),
      ],
  )


def _mk_kmid(d_in, d_out, n_al):
  return pl.pallas_call(
      functools.partial(_kmid_body, d_out),
      grid=(NPAD // BLK,),
      in_specs=[
          pl.BlockSpec((2, BLK, CH), lambda i: (0, i, 0)),
          pl.BlockSpec((1, d_in), lambda i: (0, 0)),
          pl.BlockSpec((1, d_in), lambda i: (0, 0)),
          pl.BlockSpec((1, d_in), lambda i: (0, 0)),
          pl.BlockSpec((d_in, d_out), lambda i: (0, 0)),
          pl.BlockSpec((d_out, n_al), lambda i: (0, 0)),
      ],
      out_specs=[
          pl.BlockSpec((2, BLK, CH), lambda i: (0, i, 0)),
          pl.BlockSpec((n_al, BLK), lambda i: (0, i)),
      ],
      out_shape=[
          jax.ShapeDtypeStruct((2, NPAD, CH), jnp.float32),
          jax.ShapeDtypeStruct((n_al, NPAD), jnp.float32),
      ],
  )


def _mk_k4():
  return pl.pallas_call(
      _k4_body,
      in_specs=[
          pl.BlockSpec((2, NPAD, CH), lambda: (0, 0, 0)),
          pl.BlockSpec((1, 128), lambda: (0, 0)),
          pl.BlockSpec((1, 128), lambda: (0, 0)),
          pl.BlockSpec((1, 128), lambda: (0, 0)),
          pl.BlockSpec((256, 128), lambda: (0, 0)),
          pl.BlockSpec((1, 128), lambda: (0, 0)),
      ],
      out_specs=pl.BlockSpec((1, 128), lambda: (0, 0)),
      out_shape=jax.ShapeDtypeStruct((1, 128), jnp.float32),
  )


def _sc_body(n_sub, heads_tot,
             h_hbm, alt_hbm, src_hbm, dst_hbm, gout_hbm,
             as_tab, ad_tab, src_buf, adj_buf, dst_buf, dst2_buf, grows,
             den_stage, acc, den_acc):
  sub = CH // n_sub
  nq = sub // 16
  c = lax.axis_index("c")
  s = lax.axis_index("s")
  c_n = c * NPAD

  for k in range(n_sub):
    head = (c * n_sub + k) % heads_tot
    pltpu.sync_copy(alt_hbm.at[pl.ds(head * NPAD, TBL)],
                    as_tab.at[pl.ds(k * TBL, TBL)])
    pltpu.sync_copy(alt_hbm.at[pl.ds((heads_tot + head) * NPAD, TBL)],
                    ad_tab.at[pl.ds(k * TBL, TBL)])

  base = s * ROWS_PER_SUB

  @pl.loop(0, RCH)
  def _init(i):
    r0 = base + i * RC
    pltpu.sync_copy(h_hbm.at[pl.ds(c_n + r0, RC)], grows.at[pl.ds(0, RC)])

    @pl.loop(0, RC // 16)
    def _grp(g):
      rr = r0 + g * 16
      rr = pl.multiple_of(jnp.minimum(rr, TBL - 16), 16)
      for k in range(n_sub):
        av = as_tab[pl.ds(k * TBL + rr, 16)]
        dv = ad_tab[pl.ds(k * TBL + rr, 16)]
        e = av + dv
        e = jnp.where(e >= 0, e, 0.2 * e)
        ex = jnp.exp(e)
        den_stage[k, pl.ds(g * 16, 16)] = ex
        for j in range(16):
          row = g * 16 + j
          ex_j = ex[j]
          for q in range(nq):
            sl = pl.ds(k * sub + q * 16, 16)
            grows[row, sl] = grows[row, sl] * ex_j

    pltpu.sync_copy(grows.at[pl.ds(0, RC)], acc.at[pl.ds(r0, RC)])
    for k in range(n_sub):
      pltpu.sync_copy(den_stage.at[k, pl.ds(0, RC)],
                      den_acc.at[pl.ds(k * NPAD + r0, RC)])

  plsc.subcore_barrier()

  @pl.loop(0, CPS)
  def _edges(i):
    off = (s * CPS + i) * C
    pltpu.sync_copy(src_hbm.at[pl.ds(off, C)], src_buf)
    pltpu.sync_copy(dst_hbm.at[pl.ds(off, C)], dst_buf)

    @pl.loop(0, C // 16)
    def _adj(g):
      sl = pl.ds(g * 16, 16)
      adj_buf[sl] = src_buf[sl] + c_n
      if n_sub > 1:
        dst2_buf[sl] = dst_buf[sl] + NPAD

    pltpu.sync_copy(h_hbm.at[adj_buf], grows)

    @pl.loop(0, C // 16)
    def _grp(g):
      sv = src_buf[pl.ds(g * 16, 16)]
      dv = dst_buf[pl.ds(g * 16, 16)]
      for k in range(n_sub):
        av = plsc.load_gather(as_tab, [sv + k * TBL])
        ad = plsc.load_gather(ad_tab, [dv + k * TBL])
        e = av + ad
        e = jnp.where(e >= 0, e, 0.2 * e)
        ex = jnp.exp(e)
        den_stage[k, pl.ds(g * 16, 16)] = ex
        for j in range(16):
          row = g * 16 + j
          ex_j = ex[j]
          for q in range(nq):
            sl = pl.ds(k * sub + q * 16, 16)
            grows[row, sl] = grows[row, sl] * ex_j

    pltpu.sync_copy(grows, acc.at[dst_buf], add=True)
    pltpu.sync_copy(den_stage.at[0], den_acc.at[dst_buf], add=True)
    if n_sub > 1:
      pltpu.sync_copy(den_stage.at[1], den_acc.at[dst2_buf], add=True)

  plsc.subcore_barrier()

  @pl.loop(0, RCH)
  def _norm(i):
    r0 = base + i * RC
    pltpu.sync_copy(acc.at[pl.ds(r0, RC)], grows.at[pl.ds(0, RC)])
    for k in range(n_sub):
      pltpu.sync_copy(den_acc.at[pl.ds(k * NPAD + r0, RC)],
                      den_stage.at[k, pl.ds(0, RC)])

    @pl.loop(0, RC // 16)
    def _grp(g):
      for k in range(n_sub):
        dvec = den_stage[k, pl.ds(g * 16, 16)]
        invv = jnp.float32(1.0) / dvec
        for j in range(16):
          row = g * 16 + j
          inv = invv[j]
          for q in range(nq):
            sl = pl.ds(k * sub + q * 16, 16)
            grows[row, sl] = grows[row, sl] * inv

    pltpu.sync_copy(grows.at[pl.ds(0, RC)], gout_hbm.at[c, pl.ds(r0, RC)])


def _mk_sc_agg(n_sub, heads_tot):
  mesh = plsc.VectorSubcoreMesh(core_axis_name="c", subcore_axis_name="s")
  return pl.kernel(
      functools.partial(_sc_body, n_sub, heads_tot),
      out_type=jax.ShapeDtypeStruct((NSC, NPAD, CH), jnp.float32),
      mesh=mesh,
      compiler_params=pltpu.CompilerParams(needs_layout_passes=False),
      scratch_types=[
          pltpu.VMEM((n_sub * TBL,), jnp.float32),
          pltpu.VMEM((n_sub * TBL,), jnp.float32),
          pltpu.VMEM((C,), jnp.int32),
          pltpu.VMEM((C,), jnp.int32),
          pltpu.VMEM((C,), jnp.int32),
          pltpu.VMEM((C,), jnp.int32),
          pltpu.VMEM((C, CH), jnp.float32),
          pltpu.VMEM((n_sub, C), jnp.float32),
          pltpu.VMEM_SHARED((NPAD, CH), jnp.float32),
          pltpu.VMEM_SHARED((n_sub * NPAD,), jnp.float32),
      ],
  )


def _build_a(a_s, a_d, heads, och):
  eye = jnp.eye(heads, dtype=jnp.float32)
  a_s_m = jnp.einsum("hc,hk->hck", a_s, eye).reshape(heads * och, heads)
  a_d_m = jnp.einsum("hc,hk->hck", a_d, eye).reshape(heads * och, heads)
  return jnp.concatenate([a_s_m, a_d_m], axis=1)


def kernel(x, edge_index, batch, W1, a_src1, a_dst1, b1, g1, be1,
           W2, a_src2, a_dst2, b2, g2, be2, W3, a_src3, a_dst3, b3, g3, be3,
           Wp, bp):
  del batch
  f32 = jnp.float32
  x_pad = jnp.zeros((NPAD, D_IN), f32).at[:N].set(x)
  pad_idx = jnp.full((EPAD - E,), N, jnp.int32)
  src = jnp.concatenate([edge_index[0], pad_idx])
  dst = jnp.concatenate([edge_index[1], pad_idx])

  a1 = _build_a(a_src1, a_dst1, HEADS, HID)
  a2 = _build_a(a_src2, a_dst2, HEADS, HID)
  a3 = _build_a(a_src3, a_dst3, 1, D_OUT)
  r = lambda v: jnp.reshape(v, (1, -1))

  h1, al1 = _mk_k1(D_IN, 256, 8)(x_pad, W1, a1)
  gat1 = _mk_sc_agg(2, HEADS)(
      jnp.reshape(h1, (2 * NPAD, CH)), jnp.reshape(al1, (-1,)), src, dst)
  h2, al2 = _mk_kmid(256, 256, 8)(gat1, r(b1), r(g1), r(be1), W2, a2)
  gat2 = _mk_sc_agg(2, HEADS)(
      jnp.reshape(h2, (2 * NPAD, CH)), jnp.reshape(al2, (-1,)), src, dst)
  h3, al3 = _mk_kmid(256, 128, 2)(gat2, r(b2), r(g2), r(be2), W3, a3)
  gat3 = _mk_sc_agg(1, 1)(
      jnp.reshape(h3, (2 * NPAD, CH)), jnp.reshape(al3, (-1,)), src, dst)
  out = _mk_k4()(gat3, r(b3), r(g3), r(be3), Wp, r(bp))
  return out


# speedup vs baseline: 37.1185x; 1.7983x over previous
import functools

import jax
import jax.numpy as jnp
from jax import lax
from jax.experimental import pallas as pl
from jax.experimental.pallas import tpu as pltpu
from jax.experimental.pallas import tpu_sc as plsc

N = 10000
E = 320000
D_IN = 128
HID = 64
HEADS = 4
D_OUT = 128

NSC = 2
NSUB = 16
NPAD = 10240
TBL = 10016
CH = 128
C = 32
CPS = E // (NSUB * C)
BLK = 1024
ROWS_SUB = 624


def _k1_body(x_ref, w_ref, a_ref, h_ref, al_ref):
  h = jnp.dot(x_ref[...], w_ref[...], preferred_element_type=jnp.float32)
  h_ref[0] = h[:, :CH]
  h_ref[1] = h[:, CH:]
  al_ref[...] = lax.dot_general(
      a_ref[...], h, (((0,), (1,)), ((), ())),
      preferred_element_type=jnp.float32)


def _kmid_body(d_out, gin_ref, b_ref, g_ref, be_ref, w_ref, a_ref, h_ref,
               al_ref):
  t = jnp.concatenate([gin_ref[0], gin_ref[1]], axis=-1) + b_ref[...]
  m = jnp.mean(t, axis=-1, keepdims=True)
  v = jnp.mean((t - m) ** 2, axis=-1, keepdims=True)
  t = (t - m) * lax.rsqrt(v + 1e-5) * g_ref[...] + be_ref[...]
  t = jnp.maximum(t, 0.0)
  h = jnp.dot(t, w_ref[...], preferred_element_type=jnp.float32)
  if d_out == 256:
    h_ref[0] = h[:, :CH]
    h_ref[1] = h[:, CH:]
  else:
    z = jnp.zeros_like(h[:, :64])
    h_ref[0] = jnp.concatenate([h[:, :64], z], axis=-1)
    h_ref[1] = jnp.concatenate([h[:, 64:], z], axis=-1)
  al_ref[...] = lax.dot_general(
      a_ref[...], h, (((0,), (1,)), ((), ())),
      preferred_element_type=jnp.float32)


def _k4_body(gin_ref, b_ref, g_ref, be_ref, wp_ref, bp_ref, o_ref):
  t = jnp.concatenate([gin_ref[0][:, :64], gin_ref[1][:, :64]], axis=-1)
  t = t + b_ref[...]
  m = jnp.mean(t, axis=-1, keepdims=True)
  v = jnp.mean((t - m) ** 2, axis=-1, keepdims=True)
  t = (t - m) * lax.rsqrt(v + 1e-5) * g_ref[...] + be_ref[...]
  t = jnp.maximum(t, 0.0)
  mask = lax.broadcasted_iota(jnp.int32, (NPAD, 1), 0) < N
  tm = jnp.where(mask, t, 0.0)
  mean = jnp.sum(tm, axis=0, keepdims=True) * (1.0 / N)
  tmax = jnp.max(jnp.where(mask, t, -1e30), axis=0, keepdims=True)
  feat = jnp.concatenate([mean, tmax], axis=-1)
  o_ref[...] = jnp.dot(feat, wp_ref[...],
                       preferred_element_type=jnp.float32) + bp_ref[...]


def _mk_k1(d_in, d_out, n_al):
  return pl.pallas_call(
      _k1_body,
      grid=(NPAD // BLK,),
      in_specs=[
          pl.BlockSpec((BLK, d_in), lambda i: (i, 0)),
          pl.BlockSpec((d_in, d_out), lambda i: (0, 0)),
          pl.BlockSpec((d_out, n_al), lambda i: (0, 0)),
      ],
      out_specs=[
          pl.BlockSpec((2, BLK, CH), lambda i: (0, i, 0)),
          pl.BlockSpec((n_al, BLK), lambda i: (0, i)),
      ],
      out_shape=[
          jax.ShapeDtypeStruct((2, NPAD, CH), jnp.float32),
          jax.ShapeDtypeStruct((n_al, NPAD), jnp.float32),
      ],
  )


def _mk_kmid(d_in, d_out, n_al):
  return pl.pallas_call(
      functools.partial(_kmid_body, d_out),
      grid=(NPAD // BLK,),
      in_specs=[
          pl.BlockSpec((2, BLK, CH), lambda i: (0, i, 0)),
          pl.BlockSpec((1, d_in), lambda i: (0, 0)),
          pl.BlockSpec((1, d_in), lambda i: (0, 0)),
          pl.BlockSpec((1, d_in), lambda i: (0, 0)),
          pl.BlockSpec((d_in, d_out), lambda i: (0, 0)),
          pl.BlockSpec((d_out, n_al), lambda i: (0, 0)),
      ],
      out_specs=[
          pl.BlockSpec((2, BLK, CH), lambda i: (0, i, 0)),
          pl.BlockSpec((n_al, BLK), lambda i: (0, i)),
      ],
      out_shape=[
          jax.ShapeDtypeStruct((2, NPAD, CH), jnp.float32),
          jax.ShapeDtypeStruct((n_al, NPAD), jnp.float32),
      ],
  )


def _mk_k4():
  return pl.pallas_call(
      _k4_body,
      in_specs=[
          pl.BlockSpec((2, NPAD, CH), lambda: (0, 0, 0)),
          pl.BlockSpec((1, 128), lambda: (0, 0)),
          pl.BlockSpec((1, 128), lambda: (0, 0)),
          pl.BlockSpec((1, 128), lambda: (0, 0)),
          pl.BlockSpec((256, 128), lambda: (0, 0)),
          pl.BlockSpec((1, 128), lambda: (0, 0)),
      ],
      out_specs=pl.BlockSpec((1, 128), lambda: (0, 0)),
      out_shape=jax.ShapeDtypeStruct((1, 128), jnp.float32),
  )


def _scale16(grows, den_stage, as_tab, ad_tab, sv, dv, row0, dcol0, n_sub,
             is_edge):
  sub = CH // n_sub
  nq = sub // 16
  for k in range(n_sub):
    av = plsc.load_gather(as_tab, [sv + k * TBL]) if is_edge else \
        as_tab[pl.ds(k * TBL + sv, 16)]
    ad = plsc.load_gather(ad_tab, [dv + k * TBL]) if is_edge else \
        ad_tab[pl.ds(k * TBL + sv, 16)]
    e = av + ad
    e = jnp.where(e >= 0, e, 0.2 * e)
    ex = jnp.exp(e)
    den_stage[pl.ds(dcol0 + k * C, 16)] = ex
    for j in range(16):
      row = row0 + j
      ex_j = ex[j]
      for q in range(nq):
        sl = pl.ds(k * sub + q * 16, 16)
        grows[row, sl] = grows[row, sl] * ex_j


def _sc_body(n_sub, heads_tot,
             h_hbm, alt_hbm, eidx_hbm, gout_hbm,
             as_tab, ad_tab, idx_buf, adj_buf, sidx_row, sidx_den, grows,
             den_stage, acc, den_acc, sem_idx, sem_g, sem_s):
  sub = CH // n_sub
  nq = sub // 16
  c = lax.axis_index("c")
  s = lax.axis_index("s")
  c_n = c * NPAD

  def idx_start(i, b):
    pltpu.make_async_copy(eidx_hbm.at[pl.ds(i * (2 * C), 2 * C)],
                          idx_buf.at[b], sem_idx.at[b]).start()

  def idx_wait(b):
    pltpu.make_async_copy(eidx_hbm.at[pl.ds(0, 2 * C)],
                          idx_buf.at[b], sem_idx.at[b]).wait()

  def gather_start(b):
    pltpu.make_async_copy(h_hbm.at[adj_buf.at[b]], grows.at[b],
                          sem_g.at[b]).start()

  def gather_wait(b):
    pltpu.make_async_copy(h_hbm.at[adj_buf.at[b]], grows.at[b],
                          sem_g.at[b]).wait()

  def scatter_start(b):
    pltpu.make_async_copy(grows.at[b], acc.at[sidx_row.at[b]],
                          sem_s.at[b]).start(add=True)
    pltpu.make_async_copy(den_stage.at[b], den_acc.at[sidx_den.at[b]],
                          sem_s.at[b]).start(add=True)

  def scatter_wait(b):
    pltpu.make_async_copy(grows.at[b], acc.at[sidx_row.at[b]],
                          sem_s.at[b]).wait()
    pltpu.make_async_copy(den_stage.at[b], den_acc.at[sidx_den.at[b]],
                          sem_s.at[b]).wait()

  for k in range(n_sub):
    head = (c * n_sub + k) % heads_tot
    pltpu.sync_copy(alt_hbm.at[pl.ds(head * NPAD, TBL)],
                    as_tab.at[pl.ds(k * TBL, TBL)])
    pltpu.sync_copy(alt_hbm.at[pl.ds((heads_tot + head) * NPAD, TBL)],
                    ad_tab.at[pl.ds(k * TBL, TBL)])

  base = s * ROWS_SUB
  n_rc = jnp.where(s == NSUB - 1, 41, 39)

  @pl.loop(0, n_rc)
  def _init(i):
    r0 = base + i * 16
    pltpu.sync_copy(h_hbm.at[pl.ds(c_n + r0, 16)], grows.at[0, pl.ds(0, 16)])
    _scale16(grows.at[0], den_stage.at[0], as_tab, ad_tab, r0, r0, 0, 0,
             n_sub, False)
    pltpu.sync_copy(grows.at[0, pl.ds(0, 16)], acc.at[pl.ds(r0, 16)])
    for k in range(n_sub):
      pltpu.sync_copy(den_stage.at[0, pl.ds(k * C, 16)],
                      den_acc.at[pl.ds(k * TBL + r0, 16)])

  plsc.subcore_barrier()

  q0 = s * CPS
  idx_start(q0, 0)
  idx_wait(0)

  @pl.loop(0, C // 16)
  def _adj0(g):
    sl = pl.ds(g * 16, 16)
    adj_buf[0, sl] = idx_buf[0, sl] + c_n

  gather_start(0)
  idx_start(q0 + 1, 1)

  @pl.loop(0, CPS)
  def _edges(i):
    b = i % 2
    b2 = 1 - b
    gather_wait(b)

    @pl.when(i + 1 < CPS)
    def _prefetch():
      idx_wait(b2)

      @pl.loop(0, C // 16)
      def _adj(g):
        sl = pl.ds(g * 16, 16)
        adj_buf[b2, sl] = idx_buf[b2, sl] + c_n

      @pl.when(i >= 1)
      def _():
        scatter_wait(b2)

      gather_start(b2)

    for g in range(C // 16):
      sv = idx_buf[b, pl.ds(g * 16, 16)]
      dv = idx_buf[b, pl.ds(C + g * 16, 16)]
      sidx_row[b, pl.ds(g * 16, 16)] = dv
      sidx_den[b, pl.ds(g * 16, 16)] = dv
      sidx_den[b, pl.ds(C + g * 16, 16)] = dv + TBL
      _scale16(grows.at[b], den_stage.at[b], as_tab, ad_tab, sv, dv,
               g * 16, g * 16, n_sub, True)
      if n_sub == 1:
        den_stage[b, pl.ds(C + g * 16, 16)] = den_stage[b, pl.ds(g * 16, 16)]

    scatter_start(b)

    @pl.when(i + 2 < CPS)
    def _():
      idx_start(q0 + i + 2, b)

  scatter_wait(0)
  scatter_wait(1)
  plsc.subcore_barrier()

  @pl.loop(0, n_rc)
  def _norm(i):
    r0 = base + i * 16
    pltpu.sync_copy(acc.at[pl.ds(r0, 16)], grows.at[0, pl.ds(0, 16)])
    for k in range(n_sub):
      pltpu.sync_copy(den_acc.at[pl.ds(k * TBL + r0, 16)],
                      den_stage.at[0, pl.ds(k * C, 16)])

    for k in range(n_sub):
      dvec = den_stage[0, pl.ds(k * C, 16)]
      invv = jnp.float32(1.0) / dvec
      for j in range(16):
        inv = invv[j]
        for q in range(nq):
          sl = pl.ds(k * sub + q * 16, 16)
          grows[0, j, sl] = grows[0, j, sl] * inv

    pltpu.sync_copy(grows.at[0, pl.ds(0, 16)], gout_hbm.at[c, pl.ds(r0, 16)])


def _mk_sc_agg(n_sub, heads_tot):
  mesh = plsc.VectorSubcoreMesh(core_axis_name="c", subcore_axis_name="s")
  return pl.kernel(
      functools.partial(_sc_body, n_sub, heads_tot),
      out_type=jax.ShapeDtypeStruct((NSC, NPAD, CH), jnp.float32),
      mesh=mesh,
      compiler_params=pltpu.CompilerParams(needs_layout_passes=False),
      scratch_types=[
          pltpu.VMEM((n_sub * TBL,), jnp.float32),
          pltpu.VMEM((n_sub * TBL,), jnp.float32),
          pltpu.VMEM((2, 2 * C), jnp.int32),
          pltpu.VMEM((2, C), jnp.int32),
          pltpu.VMEM((2, C), jnp.int32),
          pltpu.VMEM((2, 2 * C), jnp.int32),
          pltpu.VMEM((2, C, CH), jnp.float32),
          pltpu.VMEM((2, 2 * C), jnp.float32),
          pltpu.VMEM_SHARED((TBL, CH), jnp.float32),
          pltpu.VMEM_SHARED((2 * TBL,), jnp.float32),
          pltpu.SemaphoreType.DMA((2,)),
          pltpu.SemaphoreType.DMA((2,)),
          pltpu.SemaphoreType.DMA((2,)),
      ],
  )


def _build_a(a_s, a_d, heads, och):
  eye = jnp.eye(heads, dtype=jnp.float32)
  a_s_m = jnp.einsum("hc,hk->hck", a_s, eye).reshape(heads * och, heads)
  a_d_m = jnp.einsum("hc,hk->hck", a_d, eye).reshape(heads * och, heads)
  return jnp.concatenate([a_s_m, a_d_m], axis=1)


def kernel(x, edge_index, batch, W1, a_src1, a_dst1, b1, g1, be1,
           W2, a_src2, a_dst2, b2, g2, be2, W3, a_src3, a_dst3, b3, g3, be3,
           Wp, bp):
  del batch
  f32 = jnp.float32
  x_pad = jnp.zeros((NPAD, D_IN), f32).at[:N].set(x)
  eidx = jnp.reshape(
      jnp.transpose(jnp.reshape(edge_index, (2, E // C, C)), (1, 0, 2)), (-1,))

  a1 = _build_a(a_src1, a_dst1, HEADS, HID)
  a2 = _build_a(a_src2, a_dst2, HEADS, HID)
  a3 = _build_a(a_src3, a_dst3, 1, D_OUT)
  r = lambda v: jnp.reshape(v, (1, -1))

  h1, al1 = _mk_k1(D_IN, 256, 8)(x_pad, W1, a1)
  gat1 = _mk_sc_agg(2, HEADS)(
      jnp.reshape(h1, (2 * NPAD, CH)), jnp.reshape(al1, (-1,)), eidx)
  h2, al2 = _mk_kmid(256, 256, 8)(gat1, r(b1), r(g1), r(be1), W2, a2)
  gat2 = _mk_sc_agg(2, HEADS)(
      jnp.reshape(h2, (2 * NPAD, CH)), jnp.reshape(al2, (-1,)), eidx)
  h3, al3 = _mk_kmid(256, 128, 2)(gat2, r(b2), r(g2), r(be2), W3, a3)
  gat3 = _mk_sc_agg(1, 1)(
      jnp.reshape(h3, (2 * NPAD, CH)), jnp.reshape(al3, (-1,)), eidx)
  out = _mk_k4()(gat3, r(b3), r(g3), r(be3), Wp, r(bp))
  return out


# speedup vs baseline: 38.5113x; 1.0375x over previous
import functools

import jax
import jax.numpy as jnp
from jax import lax
from jax.experimental import pallas as pl
from jax.experimental.pallas import tpu as pltpu
from jax.experimental.pallas import tpu_sc as plsc

N = 10000
E = 320000
D_IN = 128
HID = 64
HEADS = 4
D_OUT = 128

NSC = 2
NSUB = 16
NPAD = 10240
TBL = 10016
CH = 128
C = 32
CPS = E // (NSUB * C)
BLK = 1024
ROWS_SUB = 624


def _k1_body(x_ref, w_ref, a_ref, h_ref, al_ref):
  h = jnp.dot(x_ref[...], w_ref[...], preferred_element_type=jnp.float32)
  h_ref[0] = h[:, :CH]
  h_ref[1] = h[:, CH:]
  al_ref[...] = lax.dot_general(
      a_ref[...], h, (((0,), (1,)), ((), ())),
      preferred_element_type=jnp.float32)


def _kmid_body(d_out, gin_ref, b_ref, g_ref, be_ref, w_ref, a_ref, h_ref,
               al_ref):
  t = jnp.concatenate([gin_ref[0], gin_ref[1]], axis=-1) + b_ref[...]
  m = jnp.mean(t, axis=-1, keepdims=True)
  v = jnp.mean((t - m) ** 2, axis=-1, keepdims=True)
  t = (t - m) * lax.rsqrt(v + 1e-5) * g_ref[...] + be_ref[...]
  t = jnp.maximum(t, 0.0)
  h = jnp.dot(t, w_ref[...], preferred_element_type=jnp.float32)
  if d_out == 256:
    h_ref[0] = h[:, :CH]
    h_ref[1] = h[:, CH:]
  else:
    z = jnp.zeros_like(h[:, :64])
    h_ref[0] = jnp.concatenate([h[:, :64], z], axis=-1)
    h_ref[1] = jnp.concatenate([h[:, 64:], z], axis=-1)
  al_ref[...] = lax.dot_general(
      a_ref[...], h, (((0,), (1,)), ((), ())),
      preferred_element_type=jnp.float32)


def _k4_body(gin_ref, b_ref, g_ref, be_ref, wp_ref, bp_ref, o_ref):
  t = jnp.concatenate([gin_ref[0][:, :64], gin_ref[1][:, :64]], axis=-1)
  t = t + b_ref[...]
  m = jnp.mean(t, axis=-1, keepdims=True)
  v = jnp.mean((t - m) ** 2, axis=-1, keepdims=True)
  t = (t - m) * lax.rsqrt(v + 1e-5) * g_ref[...] + be_ref[...]
  t = jnp.maximum(t, 0.0)
  mask = lax.broadcasted_iota(jnp.int32, (NPAD, 1), 0) < N
  tm = jnp.where(mask, t, 0.0)
  mean = jnp.sum(tm, axis=0, keepdims=True) * (1.0 / N)
  tmax = jnp.max(jnp.where(mask, t, -1e30), axis=0, keepdims=True)
  feat = jnp.concatenate([mean, tmax], axis=-1)
  o_ref[...] = jnp.dot(feat, wp_ref[...],
                       preferred_element_type=jnp.float32) + bp_ref[...]


def _mk_k1(d_in, d_out, n_al):
  return pl.pallas_call(
      _k1_body,
      grid=(NPAD // BLK,),
      in_specs=[
          pl.BlockSpec((BLK, d_in), lambda i: (i, 0)),
          pl.BlockSpec((d_in, d_out), lambda i: (0, 0)),
          pl.BlockSpec((d_out, n_al), lambda i: (0, 0)),
      ],
      out_specs=[
          pl.BlockSpec((2, BLK, CH), lambda i: (0, i, 0)),
          pl.BlockSpec((n_al, BLK), lambda i: (0, i)),
      ],
      out_shape=[
          jax.ShapeDtypeStruct((2, NPAD, CH), jnp.float32),
          jax.ShapeDtypeStruct((n_al, NPAD), jnp.float32),
      ],
  )


def _mk_kmid(d_in, d_out, n_al):
  return pl.pallas_call(
      functools.partial(_kmid_body, d_out),
      grid=(NPAD // BLK,),
      in_specs=[
          pl.BlockSpec((2, BLK, CH), lambda i: (0, i, 0)),
          pl.BlockSpec((1, d_in), lambda i: (0, 0)),
          pl.BlockSpec((1, d_in), lambda i: (0, 0)),
          pl.BlockSpec((1, d_in), lambda i: (0, 0)),
          pl.BlockSpec((d_in, d_out), lambda i: (0, 0)),
          pl.BlockSpec((d_out, n_al), lambda i: (0, 0)),
      ],
      out_specs=[
          pl.BlockSpec((2, BLK, CH), lambda i: (0, i, 0)),
          pl.BlockSpec((n_al, BLK), lambda i: (0, i)),
      ],
      out_shape=[
          jax.ShapeDtypeStruct((2, NPAD, CH), jnp.float32),
          jax.ShapeDtypeStruct((n_al, NPAD), jnp.float32),
      ],
  )


def _mk_k4():
  return pl.pallas_call(
      _k4_body,
      in_specs=[
          pl.BlockSpec((2, NPAD, CH), lambda: (0, 0, 0)),
          pl.BlockSpec((1, 128), lambda: (0, 0)),
          pl.BlockSpec((1, 128), lambda: (0, 0)),
          pl.BlockSpec((1, 128), lambda: (0, 0)),
          pl.BlockSpec((256, 128), lambda: (0, 0)),
          pl.BlockSpec((1, 128), lambda: (0, 0)),
      ],
      out_specs=pl.BlockSpec((1, 128), lambda: (0, 0)),
      out_shape=jax.ShapeDtypeStruct((1, 128), jnp.float32),
  )


def _scale16(grows, den_stage, as_tab, ad_tab, sv, dv, row0, dcol0, n_sub,
             is_edge):
  sub = CH // n_sub
  nq = sub // 16
  for k in range(n_sub):
    av = plsc.load_gather(as_tab, [sv + k * TBL]) if is_edge else \
        as_tab[pl.ds(k * TBL + sv, 16)]
    ad = plsc.load_gather(ad_tab, [dv + k * TBL]) if is_edge else \
        ad_tab[pl.ds(k * TBL + sv, 16)]
    e = av + ad
    e = jnp.where(e >= 0, e, 0.2 * e)
    ex = jnp.exp(e)
    den_stage[pl.ds(dcol0 + k * C, 16)] = ex
    for j in range(16):
      row = row0 + j
      ex_j = ex[j]
      for q in range(nq):
        sl = pl.ds(k * sub + q * 16, 16)
        grows[row, sl] = grows[row, sl] * ex_j


def _sc_body(n_sub, heads_tot,
             h_hbm, alt_hbm, eidx_hbm, gout_hbm,
             as_tab, ad_tab, idx_buf, adj_buf, sidx_row, sidx_den, grows,
             den_stage, acc, den_acc, sem_idx, sem_g, sem_s):
  sub = CH // n_sub
  nq = sub // 16
  c = lax.axis_index("c")
  s = lax.axis_index("s")
  c_n = c * NPAD

  def idx_start(i, b):
    pltpu.make_async_copy(eidx_hbm.at[pl.ds(i * (2 * C), 2 * C)],
                          idx_buf.at[b], sem_idx.at[b]).start()

  def idx_wait(b):
    pltpu.make_async_copy(eidx_hbm.at[pl.ds(0, 2 * C)],
                          idx_buf.at[b], sem_idx.at[b]).wait()

  def gather_start(b):
    pltpu.make_async_copy(h_hbm.at[adj_buf.at[b]], grows.at[b],
                          sem_g.at[b]).start()

  def gather_wait(b):
    pltpu.make_async_copy(h_hbm.at[adj_buf.at[b]], grows.at[b],
                          sem_g.at[b]).wait()

  def scatter_start(b):
    pltpu.make_async_copy(grows.at[b], acc.at[sidx_row.at[b]],
                          sem_s.at[b]).start(add=True)
    pltpu.make_async_copy(den_stage.at[b], den_acc.at[sidx_den.at[b]],
                          sem_s.at[b]).start(add=True)

  def scatter_wait(b):
    pltpu.make_async_copy(grows.at[b], acc.at[sidx_row.at[b]],
                          sem_s.at[b]).wait()
    pltpu.make_async_copy(den_stage.at[b], den_acc.at[sidx_den.at[b]],
                          sem_s.at[b]).wait()

  for k in range(n_sub):
    head = (c * n_sub + k) % heads_tot
    pltpu.sync_copy(alt_hbm.at[pl.ds(head * NPAD, TBL)],
                    as_tab.at[pl.ds(k * TBL, TBL)])
    pltpu.sync_copy(alt_hbm.at[pl.ds((heads_tot + head) * NPAD, TBL)],
                    ad_tab.at[pl.ds(k * TBL, TBL)])

  base = s * ROWS_SUB
  n_rc = jnp.where(s == NSUB - 1, 41, 39)

  def a_read_start(u, p):
    pltpu.make_async_copy(h_hbm.at[pl.ds(c_n + base + u * 16, 16)],
                          grows.at[p, pl.ds(0, 16)], sem_g.at[p]).start()

  def a_read_wait(p):
    pltpu.make_async_copy(h_hbm.at[pl.ds(c_n, 16)],
                          grows.at[p, pl.ds(0, 16)], sem_g.at[p]).wait()

  def a_write_start(u, p):
    r0 = base + u * 16
    pltpu.make_async_copy(grows.at[p, pl.ds(0, 16)],
                          acc.at[pl.ds(r0, 16)], sem_s.at[p]).start()
    for k in range(n_sub):
      pltpu.make_async_copy(den_stage.at[p, pl.ds(k * C, 16)],
                            den_acc.at[pl.ds(k * TBL + r0, 16)],
                            sem_s.at[p]).start()

  def a_write_wait(p):
    pltpu.make_async_copy(grows.at[p, pl.ds(0, 16)],
                          acc.at[pl.ds(base, 16)], sem_s.at[p]).wait()
    for k in range(n_sub):
      pltpu.make_async_copy(den_stage.at[p, pl.ds(k * C, 16)],
                            den_acc.at[pl.ds(k * TBL, 16)],
                            sem_s.at[p]).wait()

  a_read_start(0, 0)

  @pl.loop(0, n_rc)
  def _init(i):
    p = i % 2
    p2 = 1 - p
    a_read_wait(p)

    @pl.when(i + 1 < n_rc)
    def _():
      @pl.when(i >= 1)
      def _():
        a_write_wait(p2)

      a_read_start(i + 1, p2)

    r0 = base + i * 16
    _scale16(grows.at[p], den_stage.at[p], as_tab, ad_tab, r0, r0, 0, 0,
             n_sub, False)
    a_write_start(i, p)

  a_write_wait(0)
  a_write_wait(1)
  plsc.subcore_barrier()

  q0 = s * CPS
  idx_start(q0, 0)
  idx_wait(0)

  @pl.loop(0, C // 16)
  def _adj0(g):
    sl = pl.ds(g * 16, 16)
    adj_buf[0, sl] = idx_buf[0, sl] + c_n

  gather_start(0)
  idx_start(q0 + 1, 1)

  @pl.loop(0, CPS)
  def _edges(i):
    b = i % 2
    b2 = 1 - b
    gather_wait(b)

    @pl.when(i + 1 < CPS)
    def _prefetch():
      idx_wait(b2)

      @pl.loop(0, C // 16)
      def _adj(g):
        sl = pl.ds(g * 16, 16)
        adj_buf[b2, sl] = idx_buf[b2, sl] + c_n

      @pl.when(i >= 1)
      def _():
        scatter_wait(b2)

      gather_start(b2)

    for g in range(C // 16):
      sv = idx_buf[b, pl.ds(g * 16, 16)]
      dv = idx_buf[b, pl.ds(C + g * 16, 16)]
      sidx_row[b, pl.ds(g * 16, 16)] = dv
      sidx_den[b, pl.ds(g * 16, 16)] = dv
      sidx_den[b, pl.ds(C + g * 16, 16)] = dv + TBL
      _scale16(grows.at[b], den_stage.at[b], as_tab, ad_tab, sv, dv,
               g * 16, g * 16, n_sub, True)
      if n_sub == 1:
        den_stage[b, pl.ds(C + g * 16, 16)] = den_stage[b, pl.ds(g * 16, 16)]

    scatter_start(b)

    @pl.when(i + 2 < CPS)
    def _():
      idx_start(q0 + i + 2, b)

  scatter_wait(0)
  scatter_wait(1)
  plsc.subcore_barrier()

  def c_read_start(u, p):
    r0 = base + u * 16
    pltpu.make_async_copy(acc.at[pl.ds(r0, 16)],
                          grows.at[p, pl.ds(0, 16)], sem_g.at[p]).start()
    for k in range(n_sub):
      pltpu.make_async_copy(den_acc.at[pl.ds(k * TBL + r0, 16)],
                            den_stage.at[p, pl.ds(k * C, 16)],
                            sem_g.at[p]).start()

  def c_read_wait(p):
    pltpu.make_async_copy(acc.at[pl.ds(base, 16)],
                          grows.at[p, pl.ds(0, 16)], sem_g.at[p]).wait()
    for k in range(n_sub):
      pltpu.make_async_copy(den_acc.at[pl.ds(k * TBL, 16)],
                            den_stage.at[p, pl.ds(k * C, 16)],
                            sem_g.at[p]).wait()

  def c_write_start(u, p):
    pltpu.make_async_copy(grows.at[p, pl.ds(0, 16)],
                          gout_hbm.at[c, pl.ds(base + u * 16, 16)],
                          sem_s.at[p]).start()

  def c_write_wait(p):
    pltpu.make_async_copy(grows.at[p, pl.ds(0, 16)],
                          gout_hbm.at[c, pl.ds(base, 16)],
                          sem_s.at[p]).wait()

  c_read_start(0, 0)

  @pl.loop(0, n_rc)
  def _norm(i):
    p = i % 2
    p2 = 1 - p
    c_read_wait(p)

    @pl.when(i + 1 < n_rc)
    def _():
      @pl.when(i >= 1)
      def _():
        c_write_wait(p2)

      c_read_start(i + 1, p2)

    for k in range(n_sub):
      dvec = den_stage[p, pl.ds(k * C, 16)]
      invv = jnp.float32(1.0) / dvec
      for j in range(16):
        inv = invv[j]
        for q in range(nq):
          sl = pl.ds(k * sub + q * 16, 16)
          grows[p, j, sl] = grows[p, j, sl] * inv

    c_write_start(i, p)

  c_write_wait(0)
  c_write_wait(1)


def _mk_sc_agg(n_sub, heads_tot):
  mesh = plsc.VectorSubcoreMesh(core_axis_name="c", subcore_axis_name="s")
  return pl.kernel(
      functools.partial(_sc_body, n_sub, heads_tot),
      out_type=jax.ShapeDtypeStruct((NSC, NPAD, CH), jnp.float32),
      mesh=mesh,
      compiler_params=pltpu.CompilerParams(needs_layout_passes=False),
      scratch_types=[
          pltpu.VMEM((n_sub * TBL,), jnp.float32),
          pltpu.VMEM((n_sub * TBL,), jnp.float32),
          pltpu.VMEM((2, 2 * C), jnp.int32),
          pltpu.VMEM((2, C), jnp.int32),
          pltpu.VMEM((2, C), jnp.int32),
          pltpu.VMEM((2, 2 * C), jnp.int32),
          pltpu.VMEM((2, C, CH), jnp.float32),
          pltpu.VMEM((2, 2 * C), jnp.float32),
          pltpu.VMEM_SHARED((TBL, CH), jnp.float32),
          pltpu.VMEM_SHARED((2 * TBL,), jnp.float32),
          pltpu.SemaphoreType.DMA((2,)),
          pltpu.SemaphoreType.DMA((2,)),
          pltpu.SemaphoreType.DMA((2,)),
      ],
  )


def _build_a(a_s, a_d, heads, och):
  eye = jnp.eye(heads, dtype=jnp.float32)
  a_s_m = jnp.einsum("hc,hk->hck", a_s, eye).reshape(heads * och, heads)
  a_d_m = jnp.einsum("hc,hk->hck", a_d, eye).reshape(heads * och, heads)
  return jnp.concatenate([a_s_m, a_d_m], axis=1)


def kernel(x, edge_index, batch, W1, a_src1, a_dst1, b1, g1, be1,
           W2, a_src2, a_dst2, b2, g2, be2, W3, a_src3, a_dst3, b3, g3, be3,
           Wp, bp):
  del batch
  f32 = jnp.float32
  x_pad = jnp.zeros((NPAD, D_IN), f32).at[:N].set(x)
  eidx = jnp.reshape(
      jnp.transpose(jnp.reshape(edge_index, (2, E // C, C)), (1, 0, 2)), (-1,))

  a1 = _build_a(a_src1, a_dst1, HEADS, HID)
  a2 = _build_a(a_src2, a_dst2, HEADS, HID)
  a3 = _build_a(a_src3, a_dst3, 1, D_OUT)
  r = lambda v: jnp.reshape(v, (1, -1))

  h1, al1 = _mk_k1(D_IN, 256, 8)(x_pad, W1, a1)
  gat1 = _mk_sc_agg(2, HEADS)(
      jnp.reshape(h1, (2 * NPAD, CH)), jnp.reshape(al1, (-1,)), eidx)
  h2, al2 = _mk_kmid(256, 256, 8)(gat1, r(b1), r(g1), r(be1), W2, a2)
  gat2 = _mk_sc_agg(2, HEADS)(
      jnp.reshape(h2, (2 * NPAD, CH)), jnp.reshape(al2, (-1,)), eidx)
  h3, al3 = _mk_kmid(256, 128, 2)(gat2, r(b2), r(g2), r(be2), W3, a3)
  gat3 = _mk_sc_agg(1, 1)(
      jnp.reshape(h3, (2 * NPAD, CH)), jnp.reshape(al3, (-1,)), eidx)
  out = _mk_k4()(gat3, r(b3), r(g3), r(be3), Wp, r(bp))
  return out
